# Initial kernel scaffold; baseline (speedup 1.0000x reference)
import jax
import jax.numpy as jnp
from jax.experimental import pallas as pl


def kernel(x, g_w, gate_bias, w1, w2, w3):
    raise NotImplementedError("write your pallas kernel here")


# speedup vs baseline: 2.7171x; 2.7171x over previous
import functools

import jax
import jax.numpy as jnp
from jax.experimental import pallas as pl
from jax.experimental.pallas import tpu as pltpu

_DIM = 768
_INTER = 256
_N_SHARED = 1
_N_ROUTING = 8
_TOPK = 2
_N_EXPERTS = _N_SHARED + _N_ROUTING
_BLK_T = 256


def _moe_block_kernel(x_ref, gwt_ref, bias_ref, w1_ref, w2_ref, w3_ref, o_ref):
    xb = x_ref[...]

    scores = jnp.dot(xb, gwt_ref[...], preferred_element_type=jnp.float32)
    scores = scores - jnp.max(scores, axis=-1, keepdims=True)
    es = jnp.exp(scores)
    p = es / jnp.sum(es, axis=-1, keepdims=True)
    sel = p + bias_ref[...]

    lane = jax.lax.broadcasted_iota(jnp.int32, sel.shape, 1)
    big = jnp.int32(_N_ROUTING + 1)

    m1 = jnp.max(sel, axis=-1, keepdims=True)
    i1 = jnp.min(jnp.where(sel >= m1, lane, big), axis=-1, keepdims=True)
    oh1 = (lane == i1).astype(jnp.float32)
    sel2 = sel - oh1 * jnp.float32(1e30)
    m2 = jnp.max(sel2, axis=-1, keepdims=True)
    i2 = jnp.min(jnp.where(sel2 >= m2, lane, big), axis=-1, keepdims=True)
    oh2 = (lane == i2).astype(jnp.float32)

    p1 = jnp.sum(p * oh1, axis=-1, keepdims=True)
    p2 = jnp.sum(p * oh2, axis=-1, keepdims=True)
    wvec = p1 * oh1 + p2 * oh2

    xb16 = xb.astype(jnp.bfloat16)
    acc = jnp.zeros((_BLK_T, _DIM), jnp.float32)
    for e in range(_N_EXPERTS):
        h1 = jnp.dot(xb16, w1_ref[e], preferred_element_type=jnp.float32)
        h3 = jnp.dot(xb16, w3_ref[e], preferred_element_type=jnp.float32)
        inter = (jax.nn.silu(h1) * h3).astype(jnp.bfloat16)
        oe = jnp.dot(inter, w2_ref[e], preferred_element_type=jnp.float32)
        if e < _N_SHARED:
            acc = acc + oe
        else:
            acc = acc + wvec[:, e - _N_SHARED][:, None] * oe
    o_ref[...] = acc


@jax.jit
def kernel(x, g_w, gate_bias, w1, w2, w3):
    Bb, Tt, C = x.shape
    x2 = x.reshape(Tt, C)
    gwt = g_w.T
    bias2 = gate_bias.reshape(1, _N_ROUTING)
    w1b = w1.astype(jnp.bfloat16)
    w2b = w2.astype(jnp.bfloat16)
    w3b = w3.astype(jnp.bfloat16)

    grid = (Tt // _BLK_T,)
    out = pl.pallas_call(
        _moe_block_kernel,
        grid=grid,
        in_specs=[
            pl.BlockSpec((_BLK_T, C), lambda i: (i, 0)),
            pl.BlockSpec((C, _N_ROUTING), lambda i: (0, 0)),
            pl.BlockSpec((1, _N_ROUTING), lambda i: (0, 0)),
            pl.BlockSpec((_N_EXPERTS, C, _INTER), lambda i: (0, 0, 0)),
            pl.BlockSpec((_N_EXPERTS, _INTER, C), lambda i: (0, 0, 0)),
            pl.BlockSpec((_N_EXPERTS, C, _INTER), lambda i: (0, 0, 0)),
        ],
        out_specs=pl.BlockSpec((_BLK_T, C), lambda i: (i, 0)),
        out_shape=jax.ShapeDtypeStruct((Tt, C), jnp.float32),
    )(x2, gwt, bias2, w1b, w2b, w3b)
    return out.reshape(Bb, Tt, C)


# speedup vs baseline: 3.6830x; 1.3555x over previous
import jax
import jax.numpy as jnp
from jax.experimental import pallas as pl
from jax.experimental.pallas import tpu as pltpu

_DIM = 768
_INTER = 256
_N_SHARED = 1
_N_ROUTING = 8
_TOPK = 2
_N_EXPERTS = _N_SHARED + _N_ROUTING
_BLK_T = 512


def _moe_block_kernel(x_ref, gwt_ref, bias_ref, w1_ref, w2_ref, w3_ref, o_ref,
                      w1s, w2s, w3s):
    @pl.when(pl.program_id(0) == 0)
    def _cast_weights():
        w1s[...] = w1_ref[...].astype(jnp.bfloat16)
        w2s[...] = w2_ref[...].astype(jnp.bfloat16)
        w3s[...] = w3_ref[...].astype(jnp.bfloat16)

    xb = x_ref[...]

    scores = jnp.dot(xb, gwt_ref[...], preferred_element_type=jnp.float32)
    scores = scores - jnp.max(scores, axis=-1, keepdims=True)
    es = jnp.exp(scores)
    p = es / jnp.sum(es, axis=-1, keepdims=True)
    sel = p + bias_ref[...]

    lane = jax.lax.broadcasted_iota(jnp.int32, sel.shape, 1)
    big = jnp.int32(_N_ROUTING + 1)

    m1 = jnp.max(sel, axis=-1, keepdims=True)
    i1 = jnp.min(jnp.where(sel >= m1, lane, big), axis=-1, keepdims=True)
    oh1 = (lane == i1).astype(jnp.float32)
    sel2 = sel - oh1 * jnp.float32(1e30)
    m2 = jnp.max(sel2, axis=-1, keepdims=True)
    i2 = jnp.min(jnp.where(sel2 >= m2, lane, big), axis=-1, keepdims=True)
    oh2 = (lane == i2).astype(jnp.float32)

    p1 = jnp.sum(p * oh1, axis=-1, keepdims=True)
    p2 = jnp.sum(p * oh2, axis=-1, keepdims=True)
    wvec = p1 * oh1 + p2 * oh2

    xb16 = xb.astype(jnp.bfloat16)
    acc = jnp.zeros((_BLK_T, _DIM), jnp.float32)
    for e in range(_N_EXPERTS):
        h1 = jnp.dot(xb16, w1s[e], preferred_element_type=jnp.float32)
        h3 = jnp.dot(xb16, w3s[e], preferred_element_type=jnp.float32)
        inter = (jax.nn.silu(h1) * h3).astype(jnp.bfloat16)
        oe = jnp.dot(inter, w2s[e], preferred_element_type=jnp.float32)
        if e < _N_SHARED:
            acc = acc + oe
        else:
            acc = acc + wvec[:, e - _N_SHARED][:, None] * oe
    o_ref[...] = acc


@jax.jit
def kernel(x, g_w, gate_bias, w1, w2, w3):
    Bb, Tt, C = x.shape
    x2 = x.reshape(Tt, C)
    gwt = g_w.T
    bias2 = gate_bias.reshape(1, _N_ROUTING)

    grid = (Tt // _BLK_T,)
    out = pl.pallas_call(
        _moe_block_kernel,
        grid=grid,
        in_specs=[
            pl.BlockSpec((_BLK_T, C), lambda i: (i, 0)),
            pl.BlockSpec((C, _N_ROUTING), lambda i: (0, 0)),
            pl.BlockSpec((1, _N_ROUTING), lambda i: (0, 0)),
            pl.BlockSpec((_N_EXPERTS, C, _INTER), lambda i: (0, 0, 0)),
            pl.BlockSpec((_N_EXPERTS, _INTER, C), lambda i: (0, 0, 0)),
            pl.BlockSpec((_N_EXPERTS, C, _INTER), lambda i: (0, 0, 0)),
        ],
        out_specs=pl.BlockSpec((_BLK_T, C), lambda i: (i, 0)),
        out_shape=jax.ShapeDtypeStruct((Tt, C), jnp.float32),
        scratch_shapes=[
            pltpu.VMEM((_N_EXPERTS, _DIM, _INTER), jnp.bfloat16),
            pltpu.VMEM((_N_EXPERTS, _INTER, _DIM), jnp.bfloat16),
            pltpu.VMEM((_N_EXPERTS, _DIM, _INTER), jnp.bfloat16),
        ],
    )(x2, gwt, bias2, w1, w2, w3)
    return out.reshape(Bb, Tt, C)


# speedup vs baseline: 4.0046x; 1.0873x over previous
import jax
import jax.numpy as jnp
from jax.experimental import pallas as pl
from jax.experimental.pallas import tpu as pltpu

_DIM = 768
_INTER = 256
_N_SHARED = 1
_N_ROUTING = 8
_TOPK = 2
_N_EXPERTS = _N_SHARED + _N_ROUTING
_BLK_T = 512
_WIDE = _N_EXPERTS * _INTER


def _moe_block_kernel(x_ref, gwt_ref, bias_ref, w1_ref, w2_ref, w3_ref, o_ref,
                      w13s, w2s):
    @pl.when(pl.program_id(0) == 0)
    def _cast_weights():
        for e in range(_N_EXPERTS):
            sl = pl.ds(e * _INTER, _INTER)
            w13s[:, sl] = w1_ref[e].astype(jnp.bfloat16)
            w13s[:, pl.ds(_WIDE + e * _INTER, _INTER)] = (
                w3_ref[e].astype(jnp.bfloat16))
            w2s[sl, :] = w2_ref[e].astype(jnp.bfloat16)

    xb = x_ref[...]

    scores = jnp.dot(xb, gwt_ref[...], preferred_element_type=jnp.float32)
    scores = scores - jnp.max(scores, axis=-1, keepdims=True)
    es = jnp.exp(scores)
    p = es / jnp.sum(es, axis=-1, keepdims=True)
    sel = p + bias_ref[...]

    lane = jax.lax.broadcasted_iota(jnp.int32, sel.shape, 1)
    big = jnp.int32(_N_ROUTING + 1)

    m1 = jnp.max(sel, axis=-1, keepdims=True)
    i1 = jnp.min(jnp.where(sel >= m1, lane, big), axis=-1, keepdims=True)
    oh1 = (lane == i1).astype(jnp.float32)
    sel2 = sel - oh1 * jnp.float32(1e30)
    m2 = jnp.max(sel2, axis=-1, keepdims=True)
    i2 = jnp.min(jnp.where(sel2 >= m2, lane, big), axis=-1, keepdims=True)
    oh2 = (lane == i2).astype(jnp.float32)

    p1 = jnp.sum(p * oh1, axis=-1, keepdims=True)
    p2 = jnp.sum(p * oh2, axis=-1, keepdims=True)
    wvec = p1 * oh1 + p2 * oh2

    xb16 = xb.astype(jnp.bfloat16)
    h = jnp.dot(xb16, w13s[...], preferred_element_type=jnp.float32)
    h1 = h[:, :_WIDE]
    h3 = h[:, _WIDE:]
    inter = jax.nn.silu(h1) * h3

    pieces = [inter[:, : _N_SHARED * _INTER].astype(jnp.bfloat16)]
    for e in range(_N_ROUTING):
        lo = (_N_SHARED + e) * _INTER
        pieces.append(
            (inter[:, lo:lo + _INTER] * wvec[:, e][:, None]).astype(jnp.bfloat16))
    inter16 = jnp.concatenate(pieces, axis=1)

    o_ref[...] = jnp.dot(inter16, w2s[...], preferred_element_type=jnp.float32)


@jax.jit
def kernel(x, g_w, gate_bias, w1, w2, w3):
    Bb, Tt, C = x.shape
    x2 = x.reshape(Tt, C)
    gwt = g_w.T
    bias2 = gate_bias.reshape(1, _N_ROUTING)

    grid = (Tt // _BLK_T,)
    out = pl.pallas_call(
        _moe_block_kernel,
        grid=grid,
        in_specs=[
            pl.BlockSpec((_BLK_T, C), lambda i: (i, 0)),
            pl.BlockSpec((C, _N_ROUTING), lambda i: (0, 0)),
            pl.BlockSpec((1, _N_ROUTING), lambda i: (0, 0)),
            pl.BlockSpec((_N_EXPERTS, C, _INTER), lambda i: (0, 0, 0)),
            pl.BlockSpec((_N_EXPERTS, _INTER, C), lambda i: (0, 0, 0)),
            pl.BlockSpec((_N_EXPERTS, C, _INTER), lambda i: (0, 0, 0)),
        ],
        out_specs=pl.BlockSpec((_BLK_T, C), lambda i: (i, 0)),
        out_shape=jax.ShapeDtypeStruct((Tt, C), jnp.float32),
        scratch_shapes=[
            pltpu.VMEM((_DIM, 2 * _WIDE), jnp.bfloat16),
            pltpu.VMEM((_WIDE, _DIM), jnp.bfloat16),
        ],
    )(x2, gwt, bias2, w1, w2, w3)
    return out.reshape(Bb, Tt, C)


# speedup vs baseline: 4.0852x; 1.0201x over previous
import jax
import jax.numpy as jnp
from jax.experimental import pallas as pl
from jax.experimental.pallas import tpu as pltpu

_DIM = 768
_INTER = 256
_N_SHARED = 1
_N_ROUTING = 8
_TOPK = 2
_N_EXPERTS = _N_SHARED + _N_ROUTING
_BLK_T = 512
_WIDE = _N_EXPERTS * _INTER


def _moe_block_kernel(x_ref, gwt_ref, bias_ref, w1_ref, w2_ref, w3_ref, o_ref,
                      w13s, w2s):
    @pl.when(pl.program_id(0) == 0)
    def _cast_weights():
        for e in range(_N_EXPERTS):
            base = e * 2 * _INTER
            w13s[:, pl.ds(base, _INTER)] = w1_ref[e].astype(jnp.bfloat16)
            w13s[:, pl.ds(base + _INTER, _INTER)] = (
                w3_ref[e].astype(jnp.bfloat16))
            w2s[pl.ds(e * _INTER, _INTER), :] = w2_ref[e].astype(jnp.bfloat16)

    xb = x_ref[...]

    scores = jnp.dot(xb, gwt_ref[...], preferred_element_type=jnp.float32)
    scores = scores - jnp.max(scores, axis=-1, keepdims=True)
    es = jnp.exp(scores)
    p = es / jnp.sum(es, axis=-1, keepdims=True)
    sel = p + bias_ref[...]

    lane = jax.lax.broadcasted_iota(jnp.int32, sel.shape, 1)
    big = jnp.int32(_N_ROUTING + 1)

    m1 = jnp.max(sel, axis=-1, keepdims=True)
    i1 = jnp.min(jnp.where(sel >= m1, lane, big), axis=-1, keepdims=True)
    oh1 = (lane == i1).astype(jnp.float32)
    sel2 = sel - oh1 * jnp.float32(1e30)
    m2 = jnp.max(sel2, axis=-1, keepdims=True)
    i2 = jnp.min(jnp.where(sel2 >= m2, lane, big), axis=-1, keepdims=True)
    oh2 = (lane == i2).astype(jnp.float32)

    p1 = jnp.sum(p * oh1, axis=-1, keepdims=True)
    p2 = jnp.sum(p * oh2, axis=-1, keepdims=True)
    wvec = p1 * oh1 + p2 * oh2

    xb16 = xb.astype(jnp.bfloat16)
    pieces = []
    for e in range(_N_EXPERTS):
        he = jnp.dot(xb16, w13s[:, e * 2 * _INTER:(e + 1) * 2 * _INTER],
                     preferred_element_type=jnp.float32)
        h1e = he[:, :_INTER]
        h3e = he[:, _INTER:]
        ie = jax.nn.silu(h1e) * h3e
        if e >= _N_SHARED:
            ie = ie * wvec[:, e - _N_SHARED][:, None]
        pieces.append(ie.astype(jnp.bfloat16))
    inter16 = jnp.concatenate(pieces, axis=1)

    o_ref[...] = jnp.dot(inter16, w2s[...], preferred_element_type=jnp.float32)


@jax.jit
def kernel(x, g_w, gate_bias, w1, w2, w3):
    Bb, Tt, C = x.shape
    x2 = x.reshape(Tt, C)
    gwt = g_w.T
    bias2 = gate_bias.reshape(1, _N_ROUTING)

    grid = (Tt // _BLK_T,)
    out = pl.pallas_call(
        _moe_block_kernel,
        grid=grid,
        in_specs=[
            pl.BlockSpec((_BLK_T, C), lambda i: (i, 0)),
            pl.BlockSpec((C, _N_ROUTING), lambda i: (0, 0)),
            pl.BlockSpec((1, _N_ROUTING), lambda i: (0, 0)),
            pl.BlockSpec((_N_EXPERTS, C, _INTER), lambda i: (0, 0, 0)),
            pl.BlockSpec((_N_EXPERTS, _INTER, C), lambda i: (0, 0, 0)),
            pl.BlockSpec((_N_EXPERTS, C, _INTER), lambda i: (0, 0, 0)),
        ],
        out_specs=pl.BlockSpec((_BLK_T, C), lambda i: (i, 0)),
        out_shape=jax.ShapeDtypeStruct((Tt, C), jnp.float32),
        scratch_shapes=[
            pltpu.VMEM((_DIM, 2 * _WIDE), jnp.bfloat16),
            pltpu.VMEM((_WIDE, _DIM), jnp.bfloat16),
        ],
    )(x2, gwt, bias2, w1, w2, w3)
    return out.reshape(Bb, Tt, C)


# speedup vs baseline: 4.1112x; 1.0064x over previous
import jax
import jax.numpy as jnp
from jax.experimental import pallas as pl
from jax.experimental.pallas import tpu as pltpu

_DIM = 768
_INTER = 256
_N_SHARED = 1
_N_ROUTING = 8
_TOPK = 2
_N_EXPERTS = _N_SHARED + _N_ROUTING
_BLK_T = 1024
_WIDE = _N_EXPERTS * _INTER


def _moe_block_kernel(x_ref, gwt_ref, bias_ref, w1_ref, w2_ref, w3_ref, o_ref,
                      w13s, w2s):
    @pl.when(pl.program_id(0) == 0)
    def _cast_weights():
        for e in range(_N_EXPERTS):
            base = e * 2 * _INTER
            w13s[:, pl.ds(base, _INTER)] = w1_ref[e].astype(jnp.bfloat16)
            w13s[:, pl.ds(base + _INTER, _INTER)] = (
                w3_ref[e].astype(jnp.bfloat16))
            w2s[pl.ds(e * _INTER, _INTER), :] = w2_ref[e].astype(jnp.bfloat16)

    xb = x_ref[...]

    scores = jnp.dot(xb, gwt_ref[...], preferred_element_type=jnp.float32)
    scores = scores - jnp.max(scores, axis=-1, keepdims=True)
    es = jnp.exp(scores)
    p = es / jnp.sum(es, axis=-1, keepdims=True)
    sel = p + bias_ref[...]

    lane = jax.lax.broadcasted_iota(jnp.int32, sel.shape, 1)
    big = jnp.int32(_N_ROUTING + 1)

    m1 = jnp.max(sel, axis=-1, keepdims=True)
    i1 = jnp.min(jnp.where(sel >= m1, lane, big), axis=-1, keepdims=True)
    oh1 = (lane == i1).astype(jnp.float32)
    sel2 = sel - oh1 * jnp.float32(1e30)
    m2 = jnp.max(sel2, axis=-1, keepdims=True)
    i2 = jnp.min(jnp.where(sel2 >= m2, lane, big), axis=-1, keepdims=True)
    oh2 = (lane == i2).astype(jnp.float32)

    p1 = jnp.sum(p * oh1, axis=-1, keepdims=True)
    p2 = jnp.sum(p * oh2, axis=-1, keepdims=True)
    wvec = p1 * oh1 + p2 * oh2

    xb16 = xb.astype(jnp.bfloat16)
    pieces = []
    for e in range(_N_EXPERTS):
        he = jnp.dot(xb16, w13s[:, e * 2 * _INTER:(e + 1) * 2 * _INTER],
                     preferred_element_type=jnp.float32)
        h1e = he[:, :_INTER]
        h3e = he[:, _INTER:]
        ie = jax.nn.silu(h1e) * h3e
        if e >= _N_SHARED:
            ie = ie * wvec[:, e - _N_SHARED][:, None]
        pieces.append(ie.astype(jnp.bfloat16))
    inter16 = jnp.concatenate(pieces, axis=1)

    o_ref[...] = jnp.dot(inter16, w2s[...], preferred_element_type=jnp.float32)


@jax.jit
def kernel(x, g_w, gate_bias, w1, w2, w3):
    Bb, Tt, C = x.shape
    x2 = x.reshape(Tt, C)
    gwt = g_w.T
    bias2 = gate_bias.reshape(1, _N_ROUTING)

    grid = (Tt // _BLK_T,)
    out = pl.pallas_call(
        _moe_block_kernel,
        grid=grid,
        in_specs=[
            pl.BlockSpec((_BLK_T, C), lambda i: (i, 0)),
            pl.BlockSpec((C, _N_ROUTING), lambda i: (0, 0)),
            pl.BlockSpec((1, _N_ROUTING), lambda i: (0, 0)),
            pl.BlockSpec((_N_EXPERTS, C, _INTER), lambda i: (0, 0, 0)),
            pl.BlockSpec((_N_EXPERTS, _INTER, C), lambda i: (0, 0, 0)),
            pl.BlockSpec((_N_EXPERTS, C, _INTER), lambda i: (0, 0, 0)),
        ],
        out_specs=pl.BlockSpec((_BLK_T, C), lambda i: (i, 0)),
        out_shape=jax.ShapeDtypeStruct((Tt, C), jnp.float32),
        scratch_shapes=[
            pltpu.VMEM((_DIM, 2 * _WIDE), jnp.bfloat16),
            pltpu.VMEM((_WIDE, _DIM), jnp.bfloat16),
        ],
    )(x2, gwt, bias2, w1, w2, w3)
    return out.reshape(Bb, Tt, C)


# speedup vs baseline: 4.3057x; 1.0473x over previous
import jax
import jax.numpy as jnp
from jax.experimental import pallas as pl
from jax.experimental.pallas import tpu as pltpu

_DIM = 768
_INTER = 256
_N_SHARED = 1
_N_ROUTING = 8
_TOPK = 2
_N_EXPERTS = _N_SHARED + _N_ROUTING
_BLK_T = 1024
_WIDE = _N_EXPERTS * _INTER


def _moe_block_kernel(x_ref, gw_ref, bias_ref, w1_ref, w2_ref, w3_ref, o_ref,
                      w13s, w2s):
    @pl.when(pl.program_id(0) == 0)
    def _cast_weights():
        for e in range(_N_EXPERTS):
            base = e * 2 * _INTER
            w13s[:, pl.ds(base, _INTER)] = w1_ref[e].astype(jnp.bfloat16)
            w13s[:, pl.ds(base + _INTER, _INTER)] = (
                w3_ref[e].astype(jnp.bfloat16))
            w2s[pl.ds(e * _INTER, _INTER), :] = w2_ref[e].astype(jnp.bfloat16)

    xb = x_ref[...]

    scores = jax.lax.dot_general(
        xb, gw_ref[...], (((1,), (1,)), ((), ())),
        preferred_element_type=jnp.float32)
    scores = scores - jnp.max(scores, axis=-1, keepdims=True)
    es = jnp.exp(scores)
    p = es / jnp.sum(es, axis=-1, keepdims=True)
    sel = p + bias_ref[...]

    lane = jax.lax.broadcasted_iota(jnp.int32, sel.shape, 1)
    big = jnp.int32(_N_ROUTING + 1)

    m1 = jnp.max(sel, axis=-1, keepdims=True)
    i1 = jnp.min(jnp.where(sel >= m1, lane, big), axis=-1, keepdims=True)
    oh1 = (lane == i1).astype(jnp.float32)
    sel2 = sel - oh1 * jnp.float32(1e30)
    m2 = jnp.max(sel2, axis=-1, keepdims=True)
    i2 = jnp.min(jnp.where(sel2 >= m2, lane, big), axis=-1, keepdims=True)
    oh2 = (lane == i2).astype(jnp.float32)

    p1 = jnp.sum(p * oh1, axis=-1, keepdims=True)
    p2 = jnp.sum(p * oh2, axis=-1, keepdims=True)
    wvec = p1 * oh1 + p2 * oh2

    xb16 = xb.astype(jnp.bfloat16)
    pieces = []
    for e in range(_N_EXPERTS):
        he = jnp.dot(xb16, w13s[:, e * 2 * _INTER:(e + 1) * 2 * _INTER],
                     preferred_element_type=jnp.float32)
        h1e = he[:, :_INTER]
        h3e = he[:, _INTER:]
        ie = jax.nn.silu(h1e) * h3e
        if e >= _N_SHARED:
            ie = ie * wvec[:, e - _N_SHARED][:, None]
        pieces.append(ie.astype(jnp.bfloat16))
    inter16 = jnp.concatenate(pieces, axis=1)

    o_ref[...] = jnp.dot(inter16, w2s[...], preferred_element_type=jnp.float32)


@jax.jit
def kernel(x, g_w, gate_bias, w1, w2, w3):
    Bb, Tt, C = x.shape
    x2 = x.reshape(Tt, C)
    bias2 = gate_bias.reshape(1, _N_ROUTING)

    grid = (Tt // _BLK_T,)
    out = pl.pallas_call(
        _moe_block_kernel,
        grid=grid,
        in_specs=[
            pl.BlockSpec((_BLK_T, C), lambda i: (i, 0)),
            pl.BlockSpec((_N_ROUTING, C), lambda i: (0, 0)),
            pl.BlockSpec((1, _N_ROUTING), lambda i: (0, 0)),
            pl.BlockSpec((_N_EXPERTS, C, _INTER), lambda i: (0, 0, 0)),
            pl.BlockSpec((_N_EXPERTS, _INTER, C), lambda i: (0, 0, 0)),
            pl.BlockSpec((_N_EXPERTS, C, _INTER), lambda i: (0, 0, 0)),
        ],
        out_specs=pl.BlockSpec((_BLK_T, C), lambda i: (i, 0)),
        out_shape=jax.ShapeDtypeStruct((Tt, C), jnp.float32),
        scratch_shapes=[
            pltpu.VMEM((_DIM, 2 * _WIDE), jnp.bfloat16),
            pltpu.VMEM((_WIDE, _DIM), jnp.bfloat16),
        ],
    )(x2, g_w, bias2, w1, w2, w3)
    return out.reshape(Bb, Tt, C)
